# Initial kernel scaffold; baseline (speedup 1.0000x reference)
import jax
import jax.numpy as jnp
from jax.experimental import pallas as pl


def kernel(nfeat_ap, nfeat_sta, efeat_apap, efeat_apsta, efeat_staap, params, ei_apap, ei_apsta, ei_staap):
    raise NotImplementedError("write your pallas kernel here")


# speedup vs baseline: 4.6056x; 4.6056x over previous
import functools
import jax
import jax.numpy as jnp
from jax import lax
from jax.experimental import pallas as pl
from jax.experimental.pallas import tpu as pltpu
from jax.experimental.pallas import tpu_sc as plsc

DIM = 128
N = 5000
E = 80000
PW = 256
CH = 128
NCHUNK = E // CH
NWORKER = 32
CPW = -(-NCHUNK // NWORKER)
NPAD = 5120
ZR = NPAD // 16

_sc_mesh = plsc.VectorSubcoreMesh(core_axis_name="c", subcore_axis_name="s")


def _stats_body(x_ref, o_ref):
    i = pl.program_id(0)
    x = x_ref[...]
    blk = jnp.concatenate(
        [jnp.sum(x, axis=0, keepdims=True),
         jnp.sum(x * x, axis=0, keepdims=True)], axis=0)

    @pl.when(i == 0)
    def _():
        o_ref[...] = blk

    @pl.when(i > 0)
    def _():
        o_ref[...] += blk


def _stats(x, bs):
    r, d = x.shape
    return pl.pallas_call(
        _stats_body,
        grid=(r // bs,),
        in_specs=[pl.BlockSpec((bs, d), lambda i: (i, 0))],
        out_specs=pl.BlockSpec((2, d), lambda i: (0, 0)),
        out_shape=jax.ShapeDtypeStruct((2, d), jnp.float32),
    )(x)


def _bn_scale(stat, nrows, g, b):
    mu = stat[0] / nrows
    var = stat[1] / nrows - mu * mu
    s = g * lax.rsqrt(var + 1e-5)
    t = b - mu * s
    return s.reshape(1, -1), t.reshape(1, -1)


def _tables_body(x_ref, y_ref, sx, tx, sy, ty, wni, wnj, wnd, bnd,
                 ni_o, nj_o, ho_o):
    xb = x_ref[...] * sx[...] + tx[...]
    yb = y_ref[...] * sy[...] + ty[...]
    ni_o[...] = jnp.dot(xb, wni[...], preferred_element_type=jnp.float32)
    nj_o[...] = jnp.dot(yb, wnj[...], preferred_element_type=jnp.float32)
    ho_o[...] = jnp.dot(xb, wnd[...], preferred_element_type=jnp.float32) + bnd[...]


def _tables(x, y, sx, tx, sy, ty, wni, wnj, wnd, bnd):
    out = jax.ShapeDtypeStruct((N, DIM), jnp.float32)
    return pl.pallas_call(
        _tables_body,
        out_shape=[out, out, out],
    )(x, y, sx, tx, sy, ty, wni, wnj, wnd, bnd)


def _edge_body(g1, g2, g3, ef, se, te, wf, be, attn, fo_o, px_o):
    f = ef[...] * se[...] + te[...]
    ffij = jnp.dot(f, wf[...], preferred_element_type=jnp.float32)
    x = g1[...] + g2[...] + ffij + be[...]
    fo = jnp.where(x > 0, x, 0.01 * x)
    fo_o[...] = fo
    e = jnp.sum(fo * attn[...], axis=1, keepdims=True)
    w = jnp.exp(e)
    p = w * g3[...]
    lanes = lax.broadcasted_iota(jnp.int32, (w.shape[0], PW - DIM), 1)
    wcol = jnp.where(lanes == 0, w, 0.0)
    px_o[...] = jnp.concatenate([p, wcol], axis=1)


def _edge(g1, g2, g3, ef, se, te, wf, be, attn, bs=2000):
    d = ef.shape[1]
    grid = (E // bs,)
    bspec = lambda w_: pl.BlockSpec((bs, w_), lambda i: (i, 0))
    full = lambda a_: pl.BlockSpec(a_.shape, lambda i: tuple(0 for _ in a_.shape))
    return pl.pallas_call(
        _edge_body,
        grid=grid,
        in_specs=[bspec(DIM), bspec(DIM), bspec(DIM), bspec(d),
                  full(se), full(te), full(wf), full(be), full(attn)],
        out_specs=[bspec(DIM), bspec(PW)],
        out_shape=[jax.ShapeDtypeStruct((E, DIM), jnp.float32),
                   jax.ShapeDtypeStruct((E, PW), jnp.float32)],
    )(g1, g2, g3, ef, se, te, wf, be, attn)


def _gather3(ni_tab, nj_tab, ho_tab, src, dst):
    o = jax.ShapeDtypeStruct((E, DIM), jnp.float32)

    @functools.partial(
        pl.kernel,
        out_type=[o, o, o],
        mesh=_sc_mesh,
        scratch_types=[
            pltpu.VMEM((CH,), jnp.int32), pltpu.VMEM((CH,), jnp.int32),
            pltpu.VMEM((CH, DIM), jnp.float32),
            pltpu.VMEM((CH, DIM), jnp.float32),
            pltpu.VMEM((CH, DIM), jnp.float32),
            pltpu.SemaphoreType.DMA, pltpu.SemaphoreType.DMA,
            pltpu.SemaphoreType.DMA,
        ],
    )
    def k(ni_hbm, nj_hbm, ho_hbm, src_hbm, dst_hbm, g1_hbm, g2_hbm, g3_hbm,
          idx_s, idx_d, b1, b2, b3, s1, s2, s3):
        wid = lax.axis_index("s") * 2 + lax.axis_index("c")

        @pl.loop(0, CPW)
        def _(kk):
            c = kk * NWORKER + wid

            @pl.when(c < NCHUNK)
            def _():
                base = c * CH
                pltpu.sync_copy(src_hbm.at[pl.ds(base, CH)], idx_s)
                pltpu.sync_copy(dst_hbm.at[pl.ds(base, CH)], idx_d)
                d1 = pltpu.async_copy(ni_hbm.at[idx_s], b1, s1)
                d2 = pltpu.async_copy(nj_hbm.at[idx_d], b2, s2)
                d3 = pltpu.async_copy(ho_hbm.at[idx_s], b3, s3)
                d1.wait()
                d2.wait()
                d3.wait()
                pltpu.sync_copy(b1, g1_hbm.at[pl.ds(base, CH)])
                pltpu.sync_copy(b2, g2_hbm.at[pl.ds(base, CH)])
                pltpu.sync_copy(b3, g3_hbm.at[pl.ds(base, CH)])

    return k(ni_tab, nj_tab, ho_tab, src, dst)


def _scatter(pext, dst, zpad):
    @functools.partial(
        pl.kernel,
        out_type=jax.ShapeDtypeStruct((2, NPAD, PW), jnp.float32),
        mesh=_sc_mesh,
        compiler_params=pltpu.CompilerParams(use_tc_tiling_on_sc=False),
        scratch_types=[
            pltpu.VMEM((CH,), jnp.int32),
            pltpu.VMEM((CH, PW), jnp.float32),
            pltpu.VMEM_SHARED((NPAD, PW), jnp.float32),
        ],
    )
    def k(pext_hbm, dst_hbm, z_hbm, out_hbm, idx, buf, acc):
        cid = lax.axis_index("c")
        sid = lax.axis_index("s")
        wid = sid * 2 + cid
        pltpu.sync_copy(z_hbm.at[pl.ds(sid * ZR, ZR)],
                        acc.at[pl.ds(sid * ZR, ZR)])
        plsc.subcore_barrier()

        @pl.loop(0, CPW)
        def _(kk):
            c = kk * NWORKER + wid

            @pl.when(c < NCHUNK)
            def _():
                base = c * CH
                pltpu.sync_copy(dst_hbm.at[pl.ds(base, CH)], idx)
                pltpu.sync_copy(pext_hbm.at[pl.ds(base, CH)], buf)
                pltpu.sync_copy(buf, acc.at[idx], add=True)

        plsc.subcore_barrier()
        pltpu.sync_copy(acc.at[pl.ds(sid * ZR, ZR)],
                        out_hbm.at[cid, pl.ds(sid * ZR, ZR)])

    return k(pext, dst, zpad)


def _agg_from(ref):
    num = ref[0, :, :DIM] + ref[1, :, :DIM]
    den = ref[0, :, DIM:DIM + 1] + ref[1, :, DIM:DIM + 1]
    return jnp.where(den > 0, num / jnp.where(den > 0, den, 1.0), 0.0)


def _combine2_body(a_ref, b_ref, o_ref):
    o_ref[...] = 0.5 * (_agg_from(a_ref) + _agg_from(b_ref))


def _combine1_body(a_ref, o_ref):
    o_ref[...] = _agg_from(a_ref)


def _combine(pa, pb=None, bs=1000):
    pspec = pl.BlockSpec((2, bs, PW), lambda i: (0, i, 0))
    args = (pa,) if pb is None else (pa, pb)
    return pl.pallas_call(
        _combine1_body if pb is None else _combine2_body,
        grid=(N // bs,),
        in_specs=[pspec] * len(args),
        out_specs=pl.BlockSpec((bs, DIM), lambda i: (i, 0)),
        out_shape=jax.ShapeDtypeStruct((N, DIM), jnp.float32),
    )(*args)


def _head_body(nfs, nfa, h1, h2, w0, w1, w2, w3, cb,
               wih0, whh0, b0, wih1, whh1, b1, pw, out):
    h = (jnp.dot(nfs[...], w0[...], preferred_element_type=jnp.float32)
         + jnp.dot(nfa[...], w1[...], preferred_element_type=jnp.float32)
         + jnp.dot(h1[...], w2[...], preferred_element_type=jnp.float32)
         + jnp.dot(h2[...], w3[...], preferred_element_type=jnp.float32)
         + cb[...])
    h = jnp.maximum(h, 0.0)
    xs = [h[t * 500:(t + 1) * 500, :] for t in range(10)]
    for wih, whh, bsum in ((wih0, whh0, b0), (wih1, whh1, b1)):
        hh = jnp.zeros((500, DIM), jnp.float32)
        cc = jnp.zeros((500, DIM), jnp.float32)
        ys = []
        for t in range(10):
            g = (jnp.dot(xs[t], wih[...], preferred_element_type=jnp.float32)
                 + jnp.dot(hh, whh[...], preferred_element_type=jnp.float32)
                 + bsum[...])
            i = jax.nn.sigmoid(g[:, :DIM])
            f = jax.nn.sigmoid(g[:, DIM:2 * DIM])
            gg = jnp.tanh(g[:, 2 * DIM:3 * DIM])
            o = jax.nn.sigmoid(g[:, 3 * DIM:])
            cc = f * cc + i * gg
            hh = o * jnp.tanh(cc)
            ys.append(hh)
        xs = ys
    for t in range(10):
        z = jnp.sum(xs[t] * pw[...][:, :DIM], axis=1, keepdims=True) + pw[...][:, DIM:DIM + 1]
        out[pl.ds(t * 500, 500), :] = jnp.maximum(z, 0.0) + jnp.log1p(jnp.exp(-jnp.abs(z)))


def _head(nfs, nfa, h1, h2, w0, w1, w2, w3, cb, lstm0, lstm1, pwb):
    return pl.pallas_call(
        _head_body,
        out_shape=jax.ShapeDtypeStruct((N, 1), jnp.float32),
    )(nfs, nfa, h1, h2, w0, w1, w2, w3, cb, *lstm0, *lstm1, pwb)


def kernel(nfeat_ap, nfeat_sta, efeat_apap, efeat_apsta, efeat_staap,
           params, ei_apap, ei_apsta, ei_staap):
    p = params
    h_ap, h_sta = nfeat_ap, nfeat_sta
    efeat = {'aa': efeat_apap, 'as': efeat_apsta, 'sa': efeat_staap}
    src = {'aa': ei_apap[0], 'as': ei_apsta[0], 'sa': ei_staap[0]}
    dst = {'aa': ei_apap[1], 'as': ei_apsta[1], 'sa': ei_staap[1]}
    rel_ns = {'aa': ('a', 'a'), 'as': ('a', 's'), 'sa': ('s', 'a')}
    zpad = jnp.zeros((NPAD, PW), jnp.float32)

    hs = [nfeat_sta, nfeat_ap]
    for l in range(2):
        lp = p['layers'][l]
        st_n = {'a': _stats(h_ap, 1000), 's': _stats(h_sta, 1000)}
        sn, tn = {}, {}
        for kk in ('a', 's'):
            sn[kk], tn[kk] = _bn_scale(st_n[kk], N, lp['bn_n_g'], lp['bn_n_b'])
        node_f = {'a': h_ap, 's': h_sta}
        part = {}
        new_ef = {}
        for r, ename in (('aa', 'ap-ap'), ('as', 'ap-sta'), ('sa', 'sta-ap')):
            rp = lp[ename]
            xk, yk = rel_ns[r]
            se, te = _bn_scale(_stats(efeat[r], 2000), E,
                               lp['bn_e_g'], lp['bn_e_b'])
            ni_t, nj_t, ho_t = _tables(
                node_f[xk], node_f[yk], sn[xk], tn[xk], sn[yk], tn[yk],
                rp['W_ni'], rp['W_nj'], rp['W_node'],
                rp['b_node'].reshape(1, DIM))
            g1, g2, g3 = _gather3(ni_t, nj_t, ho_t, src[r], dst[r])
            fo, px = _edge(g1, g2, g3, efeat[r], se, te, rp['W_fij'],
                           rp['b_e'].reshape(1, DIM),
                           rp['attn'].reshape(1, DIM))
            part[r] = _scatter(px, dst[r], zpad)
            new_ef[r] = fo
        h_ap = _combine(part['aa'], part['sa'])
        h_sta = _combine(part['as'])
        efeat = new_ef
        hs.append(h_sta)

    cw = p['comb_W']
    lstm = []
    for lq in p['lstm']:
        lstm.append((lq['Wih'].T, lq['Whh'].T,
                     (lq['bih'] + lq['bhh']).reshape(1, 4 * DIM)))
    pwb = jnp.concatenate([p['pred_W'].reshape(1, DIM),
                           p['pred_b'].reshape(1, 1)], axis=1)
    return _head(hs[0], hs[1], hs[2], hs[3],
                 cw[:10], cw[10:20], cw[20:148], cw[148:276],
                 p['comb_b'].reshape(1, DIM), lstm[0], lstm[1], pwb)


# speedup vs baseline: 6.3520x; 1.3792x over previous
import functools
import jax
import jax.numpy as jnp
from jax import lax
from jax.experimental import pallas as pl
from jax.experimental.pallas import tpu as pltpu
from jax.experimental.pallas import tpu_sc as plsc

DIM = 128
N = 5000
E = 80000
CH = 128
NCHUNK = E // CH
NWORKER = 32
CPW = -(-NCHUNK // NWORKER)
NPAD = 5120
ZR = NPAD // 16

_sc_mesh = plsc.VectorSubcoreMesh(core_axis_name="c", subcore_axis_name="s")


def _stats_body(x_ref, o_ref):
    i = pl.program_id(0)
    x = x_ref[...]
    blk = jnp.concatenate(
        [jnp.sum(x, axis=0, keepdims=True),
         jnp.sum(x * x, axis=0, keepdims=True)], axis=0)

    @pl.when(i == 0)
    def _():
        o_ref[...] = blk

    @pl.when(i > 0)
    def _():
        o_ref[...] += blk


def _stats(x, bs):
    r, d = x.shape
    return pl.pallas_call(
        _stats_body,
        grid=(r // bs,),
        in_specs=[pl.BlockSpec((bs, d), lambda i: (i, 0))],
        out_specs=pl.BlockSpec((2, d), lambda i: (0, 0)),
        out_shape=jax.ShapeDtypeStruct((2, d), jnp.float32),
    )(x)


def _bn_scale(stat, nrows, g, b):
    mu = stat[0] / nrows
    var = stat[1] / nrows - mu * mu
    s = g * lax.rsqrt(var + 1e-5)
    t = b - mu * s
    return s.reshape(1, -1), t.reshape(1, -1)


def _tables_body(x_ref, y_ref, sx, tx, sy, ty, wni, wnj, wnd, bnd,
                 ni_o, nj_o, ho_o):
    xb = x_ref[...] * sx[...] + tx[...]
    yb = y_ref[...] * sy[...] + ty[...]
    ni_o[...] = jnp.dot(xb, wni[...], preferred_element_type=jnp.float32)
    nj_o[...] = jnp.dot(yb, wnj[...], preferred_element_type=jnp.float32)
    ho_o[...] = jnp.dot(xb, wnd[...], preferred_element_type=jnp.float32) + bnd[...]


def _tables(x, y, sx, tx, sy, ty, wni, wnj, wnd, bnd):
    out = jax.ShapeDtypeStruct((N, DIM), jnp.float32)
    return pl.pallas_call(
        _tables_body,
        out_shape=[out, out, out],
    )(x, y, sx, tx, sy, ty, wni, wnj, wnd, bnd)


def _edge_body(g1, g2, ef, se, te, wf, be, attn, fo_o, w_o):
    f = ef[...] * se[...] + te[...]
    ffij = jnp.dot(f, wf[...], preferred_element_type=jnp.float32)
    x = g1[...] + g2[...] + ffij + be[...]
    fo = jnp.where(x > 0, x, 0.01 * x)
    fo_o[...] = fo
    e = jnp.sum(fo * attn[...], axis=1, keepdims=True)
    w_o[...] = jnp.exp(e)


def _edge(g1, g2, ef, se, te, wf, be, attn, bs=2000):
    d = ef.shape[1]
    grid = (E // bs,)
    bspec = lambda w_: pl.BlockSpec((bs, w_), lambda i: (i, 0))
    full = lambda a_: pl.BlockSpec(a_.shape, lambda i: tuple(0 for _ in a_.shape))
    return pl.pallas_call(
        _edge_body,
        grid=grid,
        in_specs=[bspec(DIM), bspec(DIM), bspec(d),
                  full(se), full(te), full(wf), full(be), full(attn)],
        out_specs=[bspec(DIM), bspec(1)],
        out_shape=[jax.ShapeDtypeStruct((E, DIM), jnp.float32),
                   jax.ShapeDtypeStruct((E, 1), jnp.float32)],
    )(g1, g2, ef, se, te, wf, be, attn)


def _gather2(ni_tab, nj_tab, src, dst):
    o = jax.ShapeDtypeStruct((E, DIM), jnp.float32)

    @functools.partial(
        pl.kernel,
        out_type=[o, o],
        mesh=_sc_mesh,
        scratch_types=[
            pltpu.VMEM((CH,), jnp.int32), pltpu.VMEM((CH,), jnp.int32),
            pltpu.VMEM((CH, DIM), jnp.float32),
            pltpu.VMEM((CH, DIM), jnp.float32),
            pltpu.SemaphoreType.DMA, pltpu.SemaphoreType.DMA,
        ],
    )
    def k(ni_hbm, nj_hbm, src_hbm, dst_hbm, g1_hbm, g2_hbm,
          idx_s, idx_d, b1, b2, s1, s2):
        wid = lax.axis_index("s") * 2 + lax.axis_index("c")

        @pl.loop(0, CPW)
        def _(kk):
            c = kk * NWORKER + wid

            @pl.when(c < NCHUNK)
            def _():
                base = c * CH
                pltpu.sync_copy(src_hbm.at[pl.ds(base, CH)], idx_s)
                pltpu.sync_copy(dst_hbm.at[pl.ds(base, CH)], idx_d)
                d1 = pltpu.async_copy(ni_hbm.at[idx_s], b1, s1)
                d2 = pltpu.async_copy(nj_hbm.at[idx_d], b2, s2)
                d1.wait()
                d2.wait()
                pltpu.sync_copy(b1, g1_hbm.at[pl.ds(base, CH)])
                pltpu.sync_copy(b2, g2_hbm.at[pl.ds(base, CH)])

    return k(ni_tab, nj_tab, src, dst)


def _scatter(ho_tab, w, src, dst, zpad, zvec):
    @functools.partial(
        pl.kernel,
        out_type=[jax.ShapeDtypeStruct((2, NPAD, DIM), jnp.float32),
                  jax.ShapeDtypeStruct((NWORKER, NPAD), jnp.float32)],
        mesh=_sc_mesh,
        compiler_params=pltpu.CompilerParams(use_tc_tiling_on_sc=False,
                                             needs_layout_passes=False),
        scratch_types=[
            pltpu.VMEM((CH,), jnp.int32), pltpu.VMEM((CH,), jnp.int32),
            pltpu.VMEM((CH,), jnp.float32),
            pltpu.VMEM((CH, DIM), jnp.float32),
            pltpu.VMEM((NPAD,), jnp.float32),
            pltpu.VMEM_SHARED((NPAD, DIM), jnp.float32),
            pltpu.SemaphoreType.DMA,
        ],
    )
    def k(ho_hbm, w_hbm, src_hbm, dst_hbm, z_hbm, zv_hbm, out_hbm, den_hbm,
          idx_s, idx_d, wch, buf, den, acc, sem):
        cid = lax.axis_index("c")
        sid = lax.axis_index("s")
        wid = sid * 2 + cid
        pltpu.sync_copy(z_hbm.at[pl.ds(sid * ZR, ZR)],
                        acc.at[pl.ds(sid * ZR, ZR)])
        pltpu.sync_copy(zv_hbm, den)
        plsc.subcore_barrier()

        @pl.loop(0, CPW)
        def _(kk):
            c = kk * NWORKER + wid

            @pl.when(c < NCHUNK)
            def _():
                base = c * CH
                pltpu.sync_copy(src_hbm.at[pl.ds(base, CH)], idx_s)
                pltpu.sync_copy(dst_hbm.at[pl.ds(base, CH)], idx_d)
                pltpu.sync_copy(w_hbm.at[pl.ds(base, CH)], wch)
                pltpu.async_copy(ho_hbm.at[idx_s], buf, sem).wait()

                @pl.loop(0, CH)
                def _(r):
                    wv = plsc.load_gather(wch, [jnp.full((16,), r, jnp.int32)])
                    for cgrp in range(DIM // 16):
                        sl = pl.ds(cgrp * 16, 16)
                        buf[r, sl] = buf[r, sl] * wv

                @pl.loop(0, CH // 16)
                def _(j):
                    sl = pl.ds(j * 16, 16)
                    plsc.addupdate_scatter(den, [idx_d[sl]], wch[sl])

                pltpu.sync_copy(buf, acc.at[idx_d], add=True)

        plsc.subcore_barrier()
        pltpu.sync_copy(acc.at[pl.ds(sid * ZR, ZR)],
                        out_hbm.at[cid, pl.ds(sid * ZR, ZR)])
        pltpu.sync_copy(den, den_hbm.at[wid])

    return k(ho_tab, w, src, dst, zpad, zvec)


def _agg_expr(num_ref, den_ref):
    num = num_ref[0] + num_ref[1]
    den = jnp.sum(den_ref[...], axis=1, keepdims=True)
    return jnp.where(den > 0, num / jnp.where(den > 0, den, 1.0), 0.0)


def _combine2_body(na_ref, da_ref, nb_ref, db_ref, o_ref):
    o_ref[...] = 0.5 * (_agg_expr(na_ref, da_ref) + _agg_expr(nb_ref, db_ref))


def _combine1_body(na_ref, da_ref, o_ref):
    o_ref[...] = _agg_expr(na_ref, da_ref)


def _combine(pa, da, pb=None, db=None, bs=1000):
    nspec = pl.BlockSpec((2, bs, DIM), lambda i: (0, i, 0))
    dspec = pl.BlockSpec((bs, NWORKER), lambda i: (i, 0))
    if pb is None:
        args, body, specs = (pa, da), _combine1_body, [nspec, dspec]
    else:
        args, body, specs = ((pa, da, pb, db), _combine2_body,
                             [nspec, dspec, nspec, dspec])
    return pl.pallas_call(
        body,
        grid=(N // bs,),
        in_specs=specs,
        out_specs=pl.BlockSpec((bs, DIM), lambda i: (i, 0)),
        out_shape=jax.ShapeDtypeStruct((N, DIM), jnp.float32),
    )(*args)


def _head_body(nfs, nfa, h1, h2, w0, w1, w2, w3, cb,
               wih0, whh0, b0, wih1, whh1, b1, pw, out):
    h = (jnp.dot(nfs[...], w0[...], preferred_element_type=jnp.float32)
         + jnp.dot(nfa[...], w1[...], preferred_element_type=jnp.float32)
         + jnp.dot(h1[...], w2[...], preferred_element_type=jnp.float32)
         + jnp.dot(h2[...], w3[...], preferred_element_type=jnp.float32)
         + cb[...])
    h = jnp.maximum(h, 0.0)
    xs = [h[t * 500:(t + 1) * 500, :] for t in range(10)]
    for wih, whh, bsum in ((wih0, whh0, b0), (wih1, whh1, b1)):
        hh = jnp.zeros((500, DIM), jnp.float32)
        cc = jnp.zeros((500, DIM), jnp.float32)
        ys = []
        for t in range(10):
            g = (jnp.dot(xs[t], wih[...], preferred_element_type=jnp.float32)
                 + jnp.dot(hh, whh[...], preferred_element_type=jnp.float32)
                 + bsum[...])
            i = jax.nn.sigmoid(g[:, :DIM])
            f = jax.nn.sigmoid(g[:, DIM:2 * DIM])
            gg = jnp.tanh(g[:, 2 * DIM:3 * DIM])
            o = jax.nn.sigmoid(g[:, 3 * DIM:])
            cc = f * cc + i * gg
            hh = o * jnp.tanh(cc)
            ys.append(hh)
        xs = ys
    for t in range(10):
        z = jnp.sum(xs[t] * pw[...][:, :DIM], axis=1, keepdims=True) + pw[...][:, DIM:DIM + 1]
        out[pl.ds(t * 500, 500), :] = jnp.maximum(z, 0.0) + jnp.log1p(jnp.exp(-jnp.abs(z)))


def _head(nfs, nfa, h1, h2, w0, w1, w2, w3, cb, lstm0, lstm1, pwb):
    return pl.pallas_call(
        _head_body,
        out_shape=jax.ShapeDtypeStruct((N, 1), jnp.float32),
    )(nfs, nfa, h1, h2, w0, w1, w2, w3, cb, *lstm0, *lstm1, pwb)


def kernel(nfeat_ap, nfeat_sta, efeat_apap, efeat_apsta, efeat_staap,
           params, ei_apap, ei_apsta, ei_staap):
    p = params
    h_ap, h_sta = nfeat_ap, nfeat_sta
    efeat = {'aa': efeat_apap, 'as': efeat_apsta, 'sa': efeat_staap}
    src = {'aa': ei_apap[0], 'as': ei_apsta[0], 'sa': ei_staap[0]}
    dst = {'aa': ei_apap[1], 'as': ei_apsta[1], 'sa': ei_staap[1]}
    rel_ns = {'aa': ('a', 'a'), 'as': ('a', 's'), 'sa': ('s', 'a')}
    zpad = jnp.zeros((NPAD, DIM), jnp.float32)
    zvec = jnp.zeros((NPAD,), jnp.float32)

    hs = [nfeat_sta, nfeat_ap]
    for l in range(2):
        lp = p['layers'][l]
        st_n = {'a': _stats(h_ap, 1000), 's': _stats(h_sta, 1000)}
        sn, tn = {}, {}
        for kk in ('a', 's'):
            sn[kk], tn[kk] = _bn_scale(st_n[kk], N, lp['bn_n_g'], lp['bn_n_b'])
        node_f = {'a': h_ap, 's': h_sta}
        part = {}
        new_ef = {}
        for r, ename in (('aa', 'ap-ap'), ('as', 'ap-sta'), ('sa', 'sta-ap')):
            rp = lp[ename]
            xk, yk = rel_ns[r]
            se, te = _bn_scale(_stats(efeat[r], 2000), E,
                               lp['bn_e_g'], lp['bn_e_b'])
            ni_t, nj_t, ho_t = _tables(
                node_f[xk], node_f[yk], sn[xk], tn[xk], sn[yk], tn[yk],
                rp['W_ni'], rp['W_nj'], rp['W_node'],
                rp['b_node'].reshape(1, DIM))
            g1, g2 = _gather2(ni_t, nj_t, src[r], dst[r])
            fo, w2 = _edge(g1, g2, efeat[r], se, te, rp['W_fij'],
                           rp['b_e'].reshape(1, DIM),
                           rp['attn'].reshape(1, DIM))
            part[r] = _scatter(ho_t, w2.reshape(E), src[r], dst[r],
                               zpad, zvec)
            new_ef[r] = fo
        h_ap = _combine(part['aa'][0], part['aa'][1].T,
                        part['sa'][0], part['sa'][1].T)
        h_sta = _combine(part['as'][0], part['as'][1].T)
        efeat = new_ef
        hs.append(h_sta)

    cw = p['comb_W']
    lstm = []
    for lq in p['lstm']:
        lstm.append((lq['Wih'].T, lq['Whh'].T,
                     (lq['bih'] + lq['bhh']).reshape(1, 4 * DIM)))
    pwb = jnp.concatenate([p['pred_W'].reshape(1, DIM),
                           p['pred_b'].reshape(1, 1)], axis=1)
    return _head(hs[0], hs[1], hs[2], hs[3],
                 cw[:10], cw[10:20], cw[20:148], cw[148:276],
                 p['comb_b'].reshape(1, DIM), lstm[0], lstm[1], pwb)
